# baseline (speedup 1.0000x reference)
import functools

import jax
import jax.numpy as jnp
from jax import lax
from jax.experimental import pallas as pl
from jax.experimental.pallas import tpu as pltpu
from jax.experimental.pallas import tpu_sc as plsc

SEQ_LEN = 8192
EMBED_DIM = 768
BATCH = 4

ROWS_PER_CHUNK = 32
CHUNK = ROWS_PER_CHUNK * EMBED_DIM
N_WORKERS = 32


def _sc_body(in_hbm, pos_hbm, out_hbm, pos_buf, in_buf0, in_buf1,
             in_sem0, in_sem1, out_sem0, out_sem1):
    nc = 2
    wid = lax.axis_index("s") * nc + lax.axis_index("c")
    rows_per_worker = SEQ_LEN // N_WORKERS
    n_chunks = rows_per_worker // ROWS_PER_CHUNK
    seq0 = wid * rows_per_worker

    in_bufs = [in_buf0, in_buf1]
    in_sems = [in_sem0, in_sem1]
    out_sems = [out_sem0, out_sem1]

    items = [(c, b) for c in range(n_chunks) for b in range(BATCH)]

    def item_off(i):
        c, b = items[i]
        return (b * SEQ_LEN + seq0 + c * ROWS_PER_CHUNK) * EMBED_DIM

    UNROLL = 32
    n_vec = CHUNK // (16 * UNROLL)

    def make_add(buf):
        def add_body(j, _):
            base = j * (16 * UNROLL)
            for t in range(UNROLL):
                o = base + t * 16
                plsc.addupdate(buf.at[pl.ds(o, 16)], pos_buf[pl.ds(o, 16)])
            return 0
        return add_body

    pltpu.make_async_copy(
        in_hbm.at[pl.ds(item_off(0), CHUNK)], in_bufs[0], in_sems[0]).start()

    for i, (c, b) in enumerate(items):
        s = i % 2
        if b == 0:
            pltpu.sync_copy(
                pos_hbm.at[pl.ds((seq0 + c * ROWS_PER_CHUNK) * EMBED_DIM, CHUNK)],
                pos_buf)
        if i + 1 < len(items):
            ns = (i + 1) % 2
            if i >= 1:
                pltpu.make_async_copy(
                    in_bufs[ns], out_hbm.at[pl.ds(item_off(i - 1), CHUNK)],
                    out_sems[ns]).wait()
            pltpu.make_async_copy(
                in_hbm.at[pl.ds(item_off(i + 1), CHUNK)], in_bufs[ns],
                in_sems[ns]).start()
        pltpu.make_async_copy(
            in_hbm.at[pl.ds(item_off(i), CHUNK)], in_bufs[s], in_sems[s]).wait()
        lax.fori_loop(0, n_vec, make_add(in_bufs[s]), 0)
        pltpu.make_async_copy(
            in_bufs[s], out_hbm.at[pl.ds(item_off(i), CHUNK)], out_sems[s]).start()

    last = len(items) - 1
    pltpu.make_async_copy(
        in_bufs[(last - 1) % 2], out_hbm.at[pl.ds(item_off(last - 1), CHUNK)],
        out_sems[(last - 1) % 2]).wait()
    pltpu.make_async_copy(
        in_bufs[last % 2], out_hbm.at[pl.ds(item_off(last), CHUNK)],
        out_sems[last % 2]).wait()


def kernel(inputs, pos_table):
    mesh = plsc.VectorSubcoreMesh(core_axis_name="c", subcore_axis_name="s")
    k = functools.partial(
        pl.kernel,
        mesh=mesh,
        out_type=jax.ShapeDtypeStruct((BATCH * SEQ_LEN * EMBED_DIM,), jnp.float32),
        scratch_types=[
            pltpu.VMEM((CHUNK,), jnp.float32),
            pltpu.VMEM((CHUNK,), jnp.float32),
            pltpu.VMEM((CHUNK,), jnp.float32),
            pltpu.SemaphoreType.DMA,
            pltpu.SemaphoreType.DMA,
            pltpu.SemaphoreType.DMA,
            pltpu.SemaphoreType.DMA,
        ],
    )(_sc_body)
    out = k(inputs.reshape(-1), pos_table.reshape(-1))
    return out.reshape(BATCH, SEQ_LEN, EMBED_DIM)


# speedup vs baseline: 4.7738x; 4.7738x over previous
import jax
import jax.numpy as jnp
from jax.experimental import pallas as pl

SEQ_LEN = 8192
EMBED_DIM = 768
BATCH = 4

S_BLK = 256


def _add_kernel(x_ref, pos_ref, o_ref):
    o_ref[...] = x_ref[...] + pos_ref[...][None]


def kernel(inputs, pos_table):
    n_s = SEQ_LEN // S_BLK
    return pl.pallas_call(
        _add_kernel,
        grid=(n_s,),
        in_specs=[
            pl.BlockSpec((BATCH, S_BLK, EMBED_DIM), lambda s: (0, s, 0)),
            pl.BlockSpec((S_BLK, EMBED_DIM), lambda s: (s, 0)),
        ],
        out_specs=pl.BlockSpec((BATCH, S_BLK, EMBED_DIM), lambda s: (0, s, 0)),
        out_shape=jax.ShapeDtypeStruct((BATCH, SEQ_LEN, EMBED_DIM), jnp.float32),
    )(inputs, pos_table)


# speedup vs baseline: 4.9118x; 1.0289x over previous
import jax
import jax.numpy as jnp
from jax.experimental import pallas as pl

SEQ_LEN = 8192
EMBED_DIM = 768
BATCH = 4

S_BLK = 1024


def _add_kernel(x_ref, pos_ref, o_ref):
    o_ref[...] = x_ref[...] + pos_ref[...][None]


def kernel(inputs, pos_table):
    n_s = SEQ_LEN // S_BLK
    return pl.pallas_call(
        _add_kernel,
        grid=(n_s,),
        in_specs=[
            pl.BlockSpec((BATCH, S_BLK, EMBED_DIM), lambda s: (0, s, 0)),
            pl.BlockSpec((S_BLK, EMBED_DIM), lambda s: (s, 0)),
        ],
        out_specs=pl.BlockSpec((BATCH, S_BLK, EMBED_DIM), lambda s: (0, s, 0)),
        out_shape=jax.ShapeDtypeStruct((BATCH, SEQ_LEN, EMBED_DIM), jnp.float32),
    )(inputs, pos_table)
